# Initial kernel scaffold; baseline (speedup 1.0000x reference)
import jax
import jax.numpy as jnp
from jax.experimental import pallas as pl


def kernel(x, token_table, pos_table):
    raise NotImplementedError("write your pallas kernel here")


# speedup vs baseline: 1.0555x; 1.0555x over previous
import functools

import jax
import jax.numpy as jnp
from jax import lax
from jax.experimental import pallas as pl
from jax.experimental.pallas import tpu as pltpu
from jax.experimental.pallas import tpu_sc as plsc

_NC = 2
_NS = 16
_NW = _NC * _NS
_LANES = 16
_CHUNK = 40


def kernel(x, token_table, pos_table):
    B, S = x.shape
    V, E = token_table.shape
    ROWS = B * S
    RPW = ROWS // _NW
    NCH = RPW // _CHUNK

    x_w = x.reshape(_NW, NCH, _CHUNK).astype(jnp.int32)

    mesh = plsc.VectorSubcoreMesh(core_axis_name="c", subcore_axis_name="s")

    @functools.partial(
        pl.kernel,
        out_type=jax.ShapeDtypeStruct((ROWS, E), jnp.float32),
        mesh=mesh,
        scratch_types=[
            pltpu.VMEM((NCH, _CHUNK), jnp.int32),
            pltpu.VMEM((S, E), jnp.float32),
            pltpu.VMEM((_CHUNK, E), jnp.float32),
            pltpu.SemaphoreType.DMA,
        ],
        compiler_params=pltpu.CompilerParams(use_tc_tiling_on_sc=False),
    )
    def emb(x_hbm, tok_hbm, pos_hbm, out_hbm, idx_v, pos_v, buf_v, sem):
        wid = lax.axis_index("c") * _NS + lax.axis_index("s")
        pltpu.sync_copy(x_hbm.at[wid], idx_v)
        pltpu.sync_copy(pos_hbm.at[pl.ds(0, S)], pos_v)
        base = wid * RPW

        @pl.loop(0, NCH)
        def _chunk(j):
            pltpu.async_copy(tok_hbm.at[idx_v.at[j]], buf_v, sem).wait()
            poff = (j % (S // _CHUNK)) * _CHUNK

            @pl.loop(0, _CHUNK)
            def _row(r):
                for c in range(0, E, _LANES):
                    sl = pl.ds(c, _LANES)
                    buf_v[r, sl] = buf_v[r, sl] + pos_v[poff + r, sl]

            pltpu.sync_copy(buf_v, out_hbm.at[pl.ds(base + j * _CHUNK, _CHUNK)])

    out = emb(x_w, token_table, pos_table)
    return out.reshape(B, S, E)


# speedup vs baseline: 1.2354x; 1.1704x over previous
import functools

import jax
import jax.numpy as jnp
from jax import lax
from jax.experimental import pallas as pl
from jax.experimental.pallas import tpu as pltpu
from jax.experimental.pallas import tpu_sc as plsc

_NC = 2
_NS = 16
_NW = _NC * _NS
_LANES = 16
_CHUNK = 128
_NBUF = 5


def kernel(x, token_table, pos_table):
    B, S = x.shape
    V, E = token_table.shape
    ROWS = B * S
    RPW = ROWS // _NW
    NCH = RPW // _CHUNK
    NG = NCH // _NBUF

    x_w = x.reshape(_NW, NCH, _CHUNK).astype(jnp.int32)

    mesh = plsc.VectorSubcoreMesh(core_axis_name="c", subcore_axis_name="s")

    @functools.partial(
        pl.kernel,
        out_type=jax.ShapeDtypeStruct((ROWS, E), jnp.float32),
        mesh=mesh,
        scratch_types=[
            pltpu.VMEM((NCH, _CHUNK), jnp.int32),
            pltpu.VMEM((S, E), jnp.float32),
            pltpu.VMEM((_NBUF, _CHUNK, E), jnp.float32),
            pltpu.SemaphoreType.DMA((_NBUF,)),
            pltpu.SemaphoreType.DMA((_NBUF,)),
        ],
        compiler_params=pltpu.CompilerParams(use_tc_tiling_on_sc=False),
    )
    def emb(x_hbm, tok_hbm, pos_hbm, out_hbm, idx_v, pos_v, bufs, gsem, osem):
        wid = lax.axis_index("c") * _NS + lax.axis_index("s")
        pltpu.sync_copy(x_hbm.at[wid], idx_v)
        pltpu.sync_copy(pos_hbm.at[pl.ds(0, S)], pos_v)
        base = wid * RPW

        def fire_gather(m, b):
            pltpu.async_copy(tok_hbm.at[idx_v.at[m]], bufs.at[b], gsem.at[b])

        def wait_gather(b):
            pltpu.make_async_copy(
                tok_hbm.at[pl.ds(0, _CHUNK)], bufs.at[b], gsem.at[b]).wait()

        def fire_out(j, b):
            pltpu.async_copy(
                bufs.at[b], out_hbm.at[pl.ds(base + j * _CHUNK, _CHUNK)],
                osem.at[b])

        def wait_out(b):
            pltpu.make_async_copy(
                bufs.at[b], out_hbm.at[pl.ds(base, _CHUNK)], osem.at[b]).wait()

        for b in range(_NBUF):
            fire_gather(b, b)

        @pl.loop(0, NG)
        def _ring(g):
            for b in range(_NBUF):
                j = g * _NBUF + b
                bp = (b - 1) % _NBUF
                m = j + _NBUF - 1

                @pl.when(jnp.logical_and(j >= 1, m < NCH))
                def _():
                    wait_out(bp)
                    fire_gather(m, bp)

                wait_gather(b)

                poff = lax.rem(j * _CHUNK, S)
                wrapr = jnp.minimum(S - poff, _CHUNK)

                @pl.loop(0, wrapr)
                def _add_lo(r):
                    pr = poff + r
                    for c in range(0, E, _LANES):
                        sl = pl.ds(c, _LANES)
                        plsc.addupdate(bufs.at[b, r, sl], pos_v[pr, sl])

                @pl.loop(wrapr, _CHUNK)
                def _add_hi(r):
                    pr = poff + r - S
                    for c in range(0, E, _LANES):
                        sl = pl.ds(c, _LANES)
                        plsc.addupdate(bufs.at[b, r, sl], pos_v[pr, sl])

                fire_out(j, b)

        for b in range(_NBUF):
            wait_out(b)

    out = emb(x_w, token_table, pos_table)
    return out.reshape(B, S, E)


# speedup vs baseline: 1.3487x; 1.0917x over previous
import functools

import jax
import jax.numpy as jnp
from jax import lax
from jax.experimental import pallas as pl
from jax.experimental.pallas import tpu as pltpu
from jax.experimental.pallas import tpu_sc as plsc

_NC = 2
_NS = 16
_NW = _NC * _NS
_LANES = 16
_NBUF = 4


def kernel(x, token_table, pos_table):
    B, S = x.shape
    V, E = token_table.shape
    RPW = B // _NW

    mesh = plsc.VectorSubcoreMesh(core_axis_name="c", subcore_axis_name="s")

    @functools.partial(
        pl.kernel,
        out_type=jax.ShapeDtypeStruct((B, S, E), jnp.float32),
        mesh=mesh,
        scratch_types=[
            pltpu.VMEM((RPW, S), jnp.int32),
            pltpu.VMEM((S, E), jnp.float32),
            pltpu.VMEM((_NBUF, S, E), jnp.float32),
            pltpu.SemaphoreType.DMA((_NBUF,)),
            pltpu.SemaphoreType.DMA((_NBUF,)),
        ],
        compiler_params=pltpu.CompilerParams(use_tc_tiling_on_sc=False),
    )
    def emb(x_hbm, tok_hbm, pos_hbm, out_hbm, idx_v, pos_v, bufs, gsem, osem):
        wid = lax.axis_index("c") * _NS + lax.axis_index("s")
        base = wid * RPW
        pltpu.sync_copy(x_hbm.at[pl.ds(base, RPW)], idx_v)
        pltpu.sync_copy(pos_hbm.at[pl.ds(0, S)], pos_v)

        def fire_gather(m, b):
            pltpu.async_copy(tok_hbm.at[idx_v.at[m]], bufs.at[b], gsem.at[b])

        def wait_gather(b):
            pltpu.make_async_copy(
                tok_hbm.at[pl.ds(0, S)], bufs.at[b], gsem.at[b]).wait()

        def fire_out(j, b):
            pltpu.async_copy(bufs.at[b], out_hbm.at[base + j], osem.at[b])

        def wait_out(b):
            pltpu.make_async_copy(
                bufs.at[b], out_hbm.at[base], osem.at[b]).wait()

        for b in range(_NBUF):
            fire_gather(b, b)

        @pl.loop(0, RPW // _NBUF)
        def _ring(g):
            for b in range(_NBUF):
                j = g * _NBUF + b
                bp = (b - 1) % _NBUF
                m = j + _NBUF - 1

                @pl.when(jnp.logical_and(j >= 1, m < RPW))
                def _():
                    wait_out(bp)
                    fire_gather(m, bp)

                wait_gather(b)

                @pl.loop(0, S)
                def _add(r):
                    for c in range(0, E, _LANES):
                        sl = pl.ds(c, _LANES)
                        plsc.addupdate(bufs.at[b, r, sl], pos_v[r, sl])

                fire_out(j, b)

        for b in range(_NBUF):
            wait_out(b)

    return emb(x, token_table, pos_table)
